# baseline (speedup 1.0000x reference)
import functools

import jax
import jax.numpy as jnp
import numpy as np
from jax.experimental import pallas as pl

NUM_PROJ = 128
REF_SIZE = 256
D = 128
N = 1024
G3 = 768
BIG = np.float32(3e38)
INV_N = np.float32(1.0 / 1024.0)
H_GRID = np.float32(3.0 / 767.0)


def _qr_body(z0_ref, z1_ref, q0_ref, q1_ref):
    z0 = z0_ref[...]
    z1 = z1_ref[...]
    n0sq = jnp.sum(z0 * z0, axis=1, keepdims=True)
    n0 = jnp.sqrt(n0sq)
    z00 = z0[:, 0:1]
    beta = jnp.where(z00 >= 0.0, -n0, n0)
    q0 = z0 / beta
    c = jnp.sum(q0 * z1, axis=1, keepdims=True)
    r = z1 - c * q0
    rn = jnp.sqrt(jnp.sum(r * r, axis=1, keepdims=True))
    udz = jnp.sum(z0 * z1, axis=1, keepdims=True) - beta * z1[:, 0:1]
    uu = 2.0 * n0sq - 2.0 * beta * z00
    w1 = z1[:, 1:2] - 2.0 * (udz / uu) * z0[:, 1:2]
    s1 = jnp.where(w1 >= 0.0, -1.0, 1.0)
    q0_ref[...] = q0
    q1_ref[...] = s1 * (r / rn)


def _angles_body(x_ref, q0_ref, q1_ref, ang_ref):
    x = x_ref[0]
    nrm = jnp.sqrt(jnp.sum(x * x, axis=1, keepdims=True))
    xn = x / jnp.maximum(nrm, 1e-12)
    q0 = q0_ref[...]
    q1 = q1_ref[...]
    dn = (((1,), (1,)), ((), ()))
    p0 = jax.lax.dot_general(q0, xn, dn, preferred_element_type=jnp.float32)
    p1 = jax.lax.dot_general(q1, xn, dn, preferred_element_type=jnp.float32)
    pn = jnp.sqrt(p0 * p0 + p1 * p1)
    pn = jnp.maximum(pn, 1e-12)
    pn0 = p0 / pn
    pn1 = p1 / pn
    denom = -pn0
    denom = jnp.where(jnp.abs(denom) < 1e-10, 1e-10, denom)
    ang = (jnp.arctan2(-pn1, denom) + np.float32(np.pi)) / np.float32(2.0 * np.pi)
    ang_ref[0] = ang


def _embed_body(a_ref, rest_ref, int_ref, refq_ref, emb_ref):
    a2 = a_ref[0]
    ab = jnp.broadcast_to(a2.reshape(1, N), (8, N))

    m1 = jnp.min(a2)
    m2 = jnp.where(jnp.sum((a2 == m1).astype(jnp.float32)) >= 2.0, m1,
                   jnp.min(jnp.where(a2 == m1, BIG, a2)))
    M1 = jnp.max(a2)
    M2 = jnp.where(jnp.sum((a2 == M1).astype(jnp.float32)) >= 2.0, M1,
                   jnp.max(jnp.where(a2 == M1, -BIG, a2)))
    alpha = jnp.mean(a2) - 0.5

    cnts, x0s, x1s = [], [], []
    for g in range(96):
        t = rest_ref[:, g:g + 1]
        mask = ab < t
        cnts.append(jnp.sum(mask.astype(jnp.float32), axis=1, keepdims=True))
        x0s.append(jnp.max(jnp.where(mask, ab, -BIG), axis=1, keepdims=True))
        x1s.append(jnp.min(jnp.where(mask, BIG, ab), axis=1, keepdims=True))
    cnt = jnp.concatenate(cnts, axis=1)
    x0 = jnp.concatenate(x0s, axis=1)
    x1 = jnp.concatenate(x1s, axis=1)

    lo = cnt == 0.0
    hi = cnt == jnp.float32(N)
    x0 = jnp.where(lo, m1, jnp.where(hi, M2, x0))
    x1 = jnp.where(lo, m2, jnp.where(hi, M1, x1))
    y0 = jnp.clip(cnt, 1.0, jnp.float32(N - 1)) * INV_N
    rest = rest_ref[...]
    cdf = int_ref[...] + y0 + INV_N * (rest - x0) / (x1 - x0)

    cm1 = jnp.min(cdf)
    cm2 = jnp.where(jnp.sum((cdf == cm1).astype(jnp.float32)) >= 2.0, cm1,
                    jnp.min(jnp.where(cdf == cm1, BIG, cdf)))
    cM1 = jnp.max(cdf)
    cM2 = jnp.where(jnp.sum((cdf == cM1).astype(jnp.float32)) >= 2.0, cM1,
                    jnp.max(jnp.where(cdf == cM1, -BIG, cdf)))

    cdfp = jnp.concatenate([cdf, jnp.full((8, 32), BIG, jnp.float32)], axis=1)
    cdfb = jnp.broadcast_to(cdfp.reshape(1, 1024), (8, 1024))
    c2s, c0s, c1s = [], [], []
    for g in range(32):
        q = refq_ref[:, g:g + 1] - alpha
        mask = cdfb < q
        c2s.append(jnp.sum(mask.astype(jnp.float32), axis=1, keepdims=True))
        c0s.append(jnp.max(jnp.where(mask, cdfb, -BIG), axis=1, keepdims=True))
        c1s.append(jnp.min(jnp.where(mask, BIG, cdfb), axis=1, keepdims=True))
    i2 = jnp.concatenate(c2s, axis=1)
    c0 = jnp.concatenate(c0s, axis=1)
    c1 = jnp.concatenate(c1s, axis=1)

    lo2 = i2 == 0.0
    hi2 = i2 == jnp.float32(G3)
    c0 = jnp.where(lo2, cm1, jnp.where(hi2, cM2, c0))
    c1 = jnp.where(lo2, cm2, jnp.where(hi2, cM1, c1))
    ind2 = jnp.clip(i2 - 1.0, 0.0, jnp.float32(G3 - 2))
    y0q = -1.0 + H_GRID * ind2
    refq = refq_ref[...]
    emb = y0q + H_GRID * ((refq - alpha) - c0) / (c1 - c0) - refq
    emb_ref[0] = emb


def _loss_body(e1_ref, e2_ref, out_ref):
    d = jnp.abs(e2_ref[...] - e1_ref[...])
    m = jnp.minimum(d, 1.0 - d)
    s = jnp.sum(m * m, axis=(2, 3))
    loss = jnp.sqrt(s)
    out_ref[...] = jnp.broadcast_to(jnp.mean(loss, axis=1, keepdims=True), (8, 128))


def kernel(x1, x2, Z):
    f32 = jnp.float32
    Z0 = Z[:, :, 0]
    Z1 = Z[:, :, 1]

    q0, q1 = pl.pallas_call(
        _qr_body,
        out_shape=(jax.ShapeDtypeStruct((NUM_PROJ, D), f32),
                   jax.ShapeDtypeStruct((NUM_PROJ, D), f32)),
    )(Z0, Z1)

    X = jnp.stack([x1, x2]).reshape(16, 1024, D)
    ang = pl.pallas_call(
        _angles_body,
        out_shape=jax.ShapeDtypeStruct((16, NUM_PROJ, N), f32),
        grid=(16,),
        in_specs=[
            pl.BlockSpec((1, 1024, D), lambda i: (i, 0, 0)),
            pl.BlockSpec((NUM_PROJ, D), lambda i: (0, 0)),
            pl.BlockSpec((NUM_PROJ, D), lambda i: (0, 0)),
        ],
        out_specs=pl.BlockSpec((1, NUM_PROJ, N), lambda i: (i, 0, 0)),
    )(X, q0, q1)

    rows = ang.reshape(2048, 8, 128)

    xnew = jnp.linspace(-1.0, 2.0, G3).astype(f32)
    int_x = jnp.floor(xnew)
    rest_x = (xnew - int_x).reshape(8, 96)
    int_x = int_x.reshape(8, 96)
    refg = jnp.linspace(0.0, 1.0, REF_SIZE + 1)[:-1].astype(f32).reshape(8, 32)

    emb = pl.pallas_call(
        _embed_body,
        out_shape=jax.ShapeDtypeStruct((2048, 8, 32), f32),
        grid=(2048,),
        in_specs=[
            pl.BlockSpec((1, 8, 128), lambda i: (i, 0, 0)),
            pl.BlockSpec((8, 96), lambda i: (0, 0)),
            pl.BlockSpec((8, 96), lambda i: (0, 0)),
            pl.BlockSpec((8, 32), lambda i: (0, 0)),
        ],
        out_specs=pl.BlockSpec((1, 8, 32), lambda i: (i, 0, 0)),
    )(rows, rest_x, int_x, refg)

    e = emb.reshape(2, 8, NUM_PROJ, 8, 32)
    out = pl.pallas_call(
        _loss_body,
        out_shape=jax.ShapeDtypeStruct((8, 128), f32),
    )(e[0], e[1])
    return out[:, 0]


# speedup vs baseline: 14.6191x; 14.6191x over previous
import functools

import jax
import jax.numpy as jnp
import numpy as np
from jax import lax
from jax.experimental import pallas as pl
from jax.experimental.pallas import tpu as pltpu
from jax.experimental.pallas import tpu_sc as plsc

NUM_PROJ = 128
REF_SIZE = 256
D = 128
N = 1024
G3 = 768
INV_N = np.float32(1.0 / 1024.0)
H_GRID = np.float32(3.0 / 767.0)
INV_H = np.float32(767.0 / 3.0)

_xnew_np = np.linspace(-1.0, 2.0, G3).astype(np.float32)
_rest_np = _xnew_np - np.floor(_xnew_np)
_C0 = float(_rest_np[0])
_C1 = float(_rest_np[256])
_C2 = float(_rest_np[512])

f32 = jnp.float32
i32 = jnp.int32


def _qr_body(z0_ref, z1_ref, q0_ref, q1_ref):
    z0 = z0_ref[...]
    z1 = z1_ref[...]
    n0sq = jnp.sum(z0 * z0, axis=1, keepdims=True)
    n0 = jnp.sqrt(n0sq)
    z00 = z0[:, 0:1]
    beta = jnp.where(z00 >= 0.0, -n0, n0)
    q0 = z0 / beta
    c = jnp.sum(q0 * z1, axis=1, keepdims=True)
    r = z1 - c * q0
    rn = jnp.sqrt(jnp.sum(r * r, axis=1, keepdims=True))
    udz = jnp.sum(z0 * z1, axis=1, keepdims=True) - beta * z1[:, 0:1]
    uu = 2.0 * n0sq - 2.0 * beta * z00
    w1 = z1[:, 1:2] - 2.0 * (udz / uu) * z0[:, 1:2]
    s1 = jnp.where(w1 >= 0.0, -1.0, 1.0)
    q0_ref[...] = q0
    q1_ref[...] = s1 * (r / rn)


def _angles_body(x_ref, q0_ref, q1_ref, ang_ref):
    x = x_ref[0]
    nrm = jnp.sqrt(jnp.sum(x * x, axis=1, keepdims=True))
    xn = x / jnp.maximum(nrm, 1e-12)
    q0 = q0_ref[...]
    q1 = q1_ref[...]
    dn = (((1,), (1,)), ((), ()))
    p0 = lax.dot_general(q0, xn, dn, preferred_element_type=f32)
    p1 = lax.dot_general(q1, xn, dn, preferred_element_type=f32)
    pn = jnp.maximum(jnp.sqrt(p0 * p0 + p1 * p1), 1e-12)
    pn0 = p0 / pn
    pn1 = p1 / pn
    denom = -pn0
    denom = jnp.where(jnp.abs(denom) < 1e-10, 1e-10, denom)
    ang = (jnp.arctan2(-pn1, denom) + np.float32(np.pi)) / np.float32(2.0 * np.pi)
    ang_ref[0] = ang


def _ones_i32():
    return jnp.ones((16,), i32)


def _radix_pass(src, dst, pos, keyfn, shift, mask, nchunks, nbin_chunks):
    def zero(i, _):
        pos[pl.ds(i * 16, 16)] = jnp.zeros((16,), i32)
        return 0
    lax.fori_loop(0, nbin_chunks, zero, 0)

    def hist(i, _):
        v = src[pl.ds(i * 16, 16)]
        d = lax.shift_right_logical(keyfn(v), shift) & mask
        plsc.addupdate_scatter(pos, [d], _ones_i32())
        return 0
    lax.fori_loop(0, nchunks, hist, 0)

    def cum(i, c):
        h = pos[pl.ds(i * 16, 16)]
        cs = plsc.cumsum(h)
        pos[pl.ds(i * 16, 16)] = cs - h + c
        return c + jnp.sum(h)
    lax.fori_loop(0, nbin_chunks, cum, jnp.int32(0))

    def place(i, _):
        v = src[pl.ds(i * 16, 16)]
        d = lax.shift_right_logical(keyfn(v), shift) & mask
        b = plsc.load_gather(pos, [d])
        cnt, _unused = plsc.scan_count(d)
        p = b + cnt.astype(i32) - 1
        plsc.store_scatter(dst, [p], v)
        plsc.addupdate_scatter(pos, [d], _ones_i32())
        return 0
    lax.fori_loop(0, nchunks, place, 0)


def _fkey(v):
    return lax.bitcast_convert_type(v, i32)


def _ckey(v):
    b = lax.bitcast_convert_type(v, i32)
    return jnp.where(b < 0, b ^ np.int32(-1), b | np.int32(-(2 ** 31)))


def _floor_i32(x):
    return ((x + 16384.0).astype(i32)) - 16384


def _sc_embed(rows_hbm, tall_hbm, tsort_hbm, isort_hbm, refg_hbm, out_hbm,
              tall_v, tsort_v, isort_v, refg_v,
              row_v, tmp_v, srt_v, cdf_v, ctmp_v, csrt_v, emb_v, pos_v):
    wid = lax.axis_index("s") * 2 + lax.axis_index("c")
    pltpu.sync_copy(tall_hbm, tall_v)
    pltpu.sync_copy(tsort_hbm, tsort_v)
    pltpu.sync_copy(isort_hbm, isort_v)
    pltpu.sync_copy(refg_hbm, refg_v)

    def do_row(j, _):
        r = wid * 64 + j
        pltpu.sync_copy(rows_hbm.at[r], row_v)

        def acc_body(i, a):
            return a + row_v[pl.ds(i * 16, 16)]
        accv = lax.fori_loop(0, 64, acc_body, jnp.zeros((16,), f32))
        alpha = jnp.sum(accv) * np.float32(1.0 / 1024.0) - 0.5

        _radix_pass(row_v, tmp_v, pos_v, _fkey, 0, 1023, 64, 64)
        _radix_pass(tmp_v, srt_v, pos_v, _fkey, 10, 1023, 64, 64)
        _radix_pass(srt_v, tmp_v, pos_v, _fkey, 20, 1023, 64, 64)

        def zero_t(i, _):
            pos_v[pl.ds(i * 16, 16)] = jnp.zeros((16,), i32)
            return 0
        lax.fori_loop(0, 49, zero_t, 0)

        def tcnt(i, _):
            v = row_v[pl.ds(i * 16, 16)]
            k = _ones_i32()
            for off, km, cm in ((0, 256, _C0), (256, 256, _C1), (512, 255, _C2)):
                fap = _floor_i32((v - cm) * INV_H)
                fap = jnp.clip(fap + 1, 0, km)
                t0 = plsc.load_gather(tall_v, [jnp.clip(fap - 1, 0, km - 1) + off])
                t1 = plsc.load_gather(tall_v, [jnp.clip(fap, 0, km - 1) + off])
                fap = (fap
                       - ((fap >= 1) & (t0 > v)).astype(i32)
                       + ((fap <= km - 1) & (t1 <= v)).astype(i32))
                k = k + fap
            plsc.addupdate_scatter(pos_v, [k], _ones_i32())
            return 0
        lax.fori_loop(0, 64, tcnt, 0)

        def cum_t(i, c):
            h = pos_v[pl.ds(i * 16, 16)]
            cs = plsc.cumsum(h) + c
            pos_v[pl.ds(i * 16, 16)] = cs
            return c + jnp.sum(h)
        lax.fori_loop(0, 48, cum_t, jnp.int32(0))

        def cdfe(i, _):
            sl = pl.ds(i * 16, 16)
            iq = pos_v[sl]
            ind = jnp.clip(iq - 1, 0, N - 2)
            x0 = plsc.load_gather(tmp_v, [ind])
            x1 = plsc.load_gather(tmp_v, [ind + 1])
            y0 = (ind + 1).astype(f32) * INV_N
            tq = tsort_v[sl]
            cdf_v[sl] = isort_v[sl] + y0 + INV_N * (tq - x0) / (x1 - x0)
            return 0
        lax.fori_loop(0, 48, cdfe, 0)

        def zero_q(i, _):
            pos_v[pl.ds(i * 16 + 2048, 16)] = jnp.zeros((16,), i32)
            return 0
        lax.fori_loop(0, 17, zero_q, 0)

        def qcnt(i, _):
            c = cdf_v[pl.ds(i * 16, 16)]
            p = _floor_i32((c + alpha) * 256.0)
            p = jnp.clip(p + 1, 0, 256)
            r0 = plsc.load_gather(refg_v, [jnp.clip(p - 1, 0, 255)])
            r1 = plsc.load_gather(refg_v, [jnp.clip(p, 0, 255)])
            p = (p
                 - ((p >= 1) & (r0 - alpha > c)).astype(i32)
                 + ((p <= 255) & (r1 - alpha <= c)).astype(i32))
            plsc.addupdate_scatter(pos_v, [p + 2048], _ones_i32())
            return 0
        lax.fori_loop(0, 48, qcnt, 0)

        def cum_q(i, c):
            h = pos_v[pl.ds(i * 16 + 2048, 16)]
            cs = plsc.cumsum(h) + c
            pos_v[pl.ds(i * 16 + 2048, 16)] = cs
            return c + jnp.sum(h)
        lax.fori_loop(0, 16, cum_q, jnp.int32(0))

        _radix_pass(cdf_v, ctmp_v, pos_v, _ckey, 0, 2047, 48, 128)
        _radix_pass(ctmp_v, csrt_v, pos_v, _ckey, 11, 2047, 48, 128)
        _radix_pass(csrt_v, ctmp_v, pos_v, _ckey, 22, 1023, 48, 64)

        def embe(i, _):
            sl = pl.ds(i * 16, 16)
            i2 = pos_v[pl.ds(i * 16 + 2048, 16)]
            ind = jnp.clip(i2 - 1, 0, G3 - 2)
            c0 = plsc.load_gather(ctmp_v, [ind])
            c1 = plsc.load_gather(ctmp_v, [ind + 1])
            rq = refg_v[sl]
            t = rq - alpha
            y0 = -1.0 + H_GRID * ind.astype(f32)
            emb_v[sl] = y0 + H_GRID * (t - c0) / (c1 - c0) - rq
            return 0
        lax.fori_loop(0, 16, embe, 0)
        pltpu.sync_copy(emb_v, out_hbm.at[r])
        return 0

    lax.fori_loop(0, 64, do_row, 0)


_sc_call = functools.partial(
    pl.kernel,
    out_type=jax.ShapeDtypeStruct((2048, REF_SIZE), f32),
    mesh=plsc.VectorSubcoreMesh(core_axis_name="c", subcore_axis_name="s"),
    compiler_params=pltpu.CompilerParams(needs_layout_passes=False),
    scratch_types=[
        pltpu.VMEM((G3,), f32),
        pltpu.VMEM((G3,), f32),
        pltpu.VMEM((G3,), f32),
        pltpu.VMEM((REF_SIZE,), f32),
        pltpu.VMEM((N,), f32),
        pltpu.VMEM((N,), f32),
        pltpu.VMEM((N,), f32),
        pltpu.VMEM((G3,), f32),
        pltpu.VMEM((G3,), f32),
        pltpu.VMEM((G3,), f32),
        pltpu.VMEM((REF_SIZE,), f32),
        pltpu.VMEM((2048 + 272,), i32),
    ],
)(_sc_embed)


def _loss_body(e1_ref, e2_ref, out_ref):
    d = jnp.abs(e2_ref[...] - e1_ref[...])
    m = jnp.minimum(d, 1.0 - d)
    s = jnp.sum(m * m, axis=2)
    loss = jnp.sqrt(s)
    out_ref[...] = jnp.broadcast_to(jnp.mean(loss, axis=1, keepdims=True), (8, 128))


def kernel(x1, x2, Z):
    Z0 = Z[:, :, 0]
    Z1 = Z[:, :, 1]

    q0, q1 = pl.pallas_call(
        _qr_body,
        out_shape=(jax.ShapeDtypeStruct((NUM_PROJ, D), f32),
                   jax.ShapeDtypeStruct((NUM_PROJ, D), f32)),
    )(Z0, Z1)

    X = jnp.stack([x1, x2]).reshape(16, 1024, D)
    ang = pl.pallas_call(
        _angles_body,
        out_shape=jax.ShapeDtypeStruct((16, NUM_PROJ, N), f32),
        grid=(16,),
        in_specs=[
            pl.BlockSpec((1, 1024, D), lambda i: (i, 0, 0)),
            pl.BlockSpec((NUM_PROJ, D), lambda i: (0, 0)),
            pl.BlockSpec((NUM_PROJ, D), lambda i: (0, 0)),
        ],
        out_specs=pl.BlockSpec((1, NUM_PROJ, N), lambda i: (i, 0, 0)),
    )(X, q0, q1)

    rows = ang.reshape(2048, N)

    xnew = jnp.linspace(-1.0, 2.0, G3).astype(f32)
    int_x = jnp.floor(xnew)
    rest_x = xnew - int_x
    perm = jnp.argsort(rest_x, stable=True)
    tsort = rest_x[perm]
    isort = int_x[perm]
    tall = jnp.concatenate([rest_x[:767], jnp.full((1,), 9.0, f32)])
    refg = jnp.linspace(0.0, 1.0, REF_SIZE + 1)[:-1].astype(f32)

    emb = _sc_call(rows, tall, tsort, isort, refg)

    e = emb.reshape(2, 8, NUM_PROJ, REF_SIZE)
    out = pl.pallas_call(
        _loss_body,
        out_shape=jax.ShapeDtypeStruct((8, 128), f32),
    )(e[0], e[1])
    return out[:, 0]


# speedup vs baseline: 19.6855x; 1.3466x over previous
import functools

import jax
import jax.numpy as jnp
import numpy as np
from jax import lax
from jax.experimental import pallas as pl
from jax.experimental.pallas import tpu as pltpu
from jax.experimental.pallas import tpu_sc as plsc

NUM_PROJ = 128
REF_SIZE = 256
D = 128
N = 1024
G3 = 768
INV_N = np.float32(1.0 / 1024.0)
H_GRID = np.float32(3.0 / 767.0)

f32 = jnp.float32
i32 = jnp.int32


def _qr_body(z0_ref, z1_ref, q0_ref, q1_ref):
    z0 = z0_ref[...]
    z1 = z1_ref[...]
    n0sq = jnp.sum(z0 * z0, axis=1, keepdims=True)
    n0 = jnp.sqrt(n0sq)
    z00 = z0[:, 0:1]
    beta = jnp.where(z00 >= 0.0, -n0, n0)
    q0 = z0 / beta
    c = jnp.sum(q0 * z1, axis=1, keepdims=True)
    r = z1 - c * q0
    rn = jnp.sqrt(jnp.sum(r * r, axis=1, keepdims=True))
    udz = jnp.sum(z0 * z1, axis=1, keepdims=True) - beta * z1[:, 0:1]
    uu = 2.0 * n0sq - 2.0 * beta * z00
    w1 = z1[:, 1:2] - 2.0 * (udz / uu) * z0[:, 1:2]
    s1 = jnp.where(w1 >= 0.0, -1.0, 1.0)
    q0_ref[...] = q0
    q1_ref[...] = s1 * (r / rn)


def _angles_body(x_ref, q0_ref, q1_ref, ang_ref):
    x = x_ref[0]
    nrm = jnp.sqrt(jnp.sum(x * x, axis=1, keepdims=True))
    xn = x / jnp.maximum(nrm, 1e-12)
    q0 = q0_ref[...]
    q1 = q1_ref[...]
    dn = (((1,), (1,)), ((), ()))
    p0 = lax.dot_general(q0, xn, dn, preferred_element_type=f32)
    p1 = lax.dot_general(q1, xn, dn, preferred_element_type=f32)
    pn = jnp.maximum(jnp.sqrt(p0 * p0 + p1 * p1), 1e-12)
    pn0 = p0 / pn
    pn1 = p1 / pn
    denom = -pn0
    denom = jnp.where(jnp.abs(denom) < 1e-10, 1e-10, denom)
    ang = (jnp.arctan2(-pn1, denom) + np.float32(np.pi)) / np.float32(2.0 * np.pi)
    ang_ref[0] = ang


def _ones_i32():
    return jnp.ones((16,), i32)


def _radix_pass(src, dst, pos, keyfn, shift, mask, nchunks, nbin_chunks):
    def zero(i, _):
        pos[pl.ds(i * 16, 16)] = jnp.zeros((16,), i32)
        return 0
    lax.fori_loop(0, nbin_chunks, zero, 0, unroll=4)

    def hist(i, _):
        v = src[pl.ds(i * 16, 16)]
        d = lax.shift_right_logical(keyfn(v), shift) & mask
        plsc.addupdate_scatter(pos, [d], _ones_i32())
        return 0
    lax.fori_loop(0, nchunks, hist, 0, unroll=4)

    def cum(i, c):
        h = pos[pl.ds(i * 16, 16)]
        cs = plsc.cumsum(h)
        pos[pl.ds(i * 16, 16)] = cs - h + c
        return c + jnp.sum(h)
    lax.fori_loop(0, nbin_chunks, cum, jnp.int32(0), unroll=2)

    def place(i, _):
        v = src[pl.ds(i * 16, 16)]
        d = lax.shift_right_logical(keyfn(v), shift) & mask
        b = plsc.load_gather(pos, [d])
        cnt, _unused = plsc.scan_count(d)
        p = b + cnt.astype(i32) - 1
        plsc.store_scatter(dst, [p], v)
        plsc.addupdate_scatter(pos, [d], _ones_i32())
        return 0
    lax.fori_loop(0, nchunks, place, 0, unroll=2)


def _fkey(v):
    return lax.bitcast_convert_type(v, i32)


def _qkey(v):
    k = jnp.clip((v + 2.0) * 204800.0, 0.0, 1048575.0)
    return k.astype(i32)


def _floor_i32(x):
    return ((x + 16384.0).astype(i32)) - 16384


def _sc_embed(rows_hbm, tsort_hbm, isort_hbm, refg_hbm, out_hbm,
              tsort_v, isort_v, refg_v,
              row_v, tmp_v, srt_v, cdf_v, ctmp_v, csrt_v, emb_v, pos_v):
    wid = lax.axis_index("s") * 2 + lax.axis_index("c")
    pltpu.sync_copy(tsort_hbm, tsort_v)
    pltpu.sync_copy(isort_hbm, isort_v)
    pltpu.sync_copy(refg_hbm, refg_v)

    def do_row(j, _):
        r = wid * 64 + j
        pltpu.sync_copy(rows_hbm.at[r], row_v)

        def acc_body(i, a):
            return a + row_v[pl.ds(i * 16, 16)]
        accv = lax.fori_loop(0, 64, acc_body, jnp.zeros((16,), f32), unroll=4)
        alpha = jnp.sum(accv) * np.float32(1.0 / 1024.0) - 0.5

        _radix_pass(row_v, tmp_v, pos_v, _fkey, 0, 1023, 64, 64)
        _radix_pass(tmp_v, srt_v, pos_v, _fkey, 10, 1023, 64, 64)
        _radix_pass(srt_v, tmp_v, pos_v, _fkey, 20, 1023, 64, 64)

        def zero_t(i, _):
            pos_v[pl.ds(i * 16, 16)] = jnp.zeros((16,), i32)
            return 0
        lax.fori_loop(0, 49, zero_t, 0, unroll=4)

        def tcnt(i, _):
            v = row_v[pl.ds(i * 16, 16)]
            k = jnp.clip(_floor_i32(v * 767.0) + 2, 2, G3)
            t0 = plsc.load_gather(tsort_v, [k - 1])
            t1 = plsc.load_gather(tsort_v, [jnp.minimum(k, G3 - 1)])
            k = (k
                 - ((k >= 3) & (t0 > v)).astype(i32)
                 + ((k <= G3 - 1) & (t1 <= v)).astype(i32))
            plsc.addupdate_scatter(pos_v, [k], _ones_i32())
            return 0
        lax.fori_loop(0, 64, tcnt, 0, unroll=4)

        def cum_t(i, c):
            h = pos_v[pl.ds(i * 16, 16)]
            cs = plsc.cumsum(h) + c
            pos_v[pl.ds(i * 16, 16)] = cs
            return c + jnp.sum(h)
        lax.fori_loop(0, 48, cum_t, jnp.int32(0), unroll=2)

        def cdfe(i, _):
            sl = pl.ds(i * 16, 16)
            iq = pos_v[sl]
            ind = jnp.clip(iq - 1, 0, N - 2)
            x0 = plsc.load_gather(tmp_v, [ind])
            x1 = plsc.load_gather(tmp_v, [ind + 1])
            y0 = (ind + 1).astype(f32) * INV_N
            tq = tsort_v[sl]
            cdf_v[sl] = isort_v[sl] + y0 + INV_N * (tq - x0) / (x1 - x0)
            return 0
        lax.fori_loop(0, 48, cdfe, 0, unroll=4)

        def zero_q(i, _):
            pos_v[pl.ds(i * 16 + 2048, 16)] = jnp.zeros((16,), i32)
            return 0
        lax.fori_loop(0, 17, zero_q, 0, unroll=4)

        def qcnt(i, _):
            c = cdf_v[pl.ds(i * 16, 16)]
            p = _floor_i32((c + alpha) * 256.0)
            p = jnp.clip(p + 1, 0, 256)
            r0 = plsc.load_gather(refg_v, [jnp.clip(p - 1, 0, 255)])
            r1 = plsc.load_gather(refg_v, [jnp.clip(p, 0, 255)])
            p = (p
                 - ((p >= 1) & (r0 - alpha > c)).astype(i32)
                 + ((p <= 255) & (r1 - alpha <= c)).astype(i32))
            plsc.addupdate_scatter(pos_v, [p + 2048], _ones_i32())
            return 0
        lax.fori_loop(0, 48, qcnt, 0, unroll=4)

        def cum_q(i, c):
            h = pos_v[pl.ds(i * 16 + 2048, 16)]
            cs = plsc.cumsum(h) + c
            pos_v[pl.ds(i * 16 + 2048, 16)] = cs
            return c + jnp.sum(h)
        lax.fori_loop(0, 16, cum_q, jnp.int32(0), unroll=2)

        _radix_pass(cdf_v, ctmp_v, pos_v, _qkey, 0, 1023, 48, 64)
        _radix_pass(ctmp_v, csrt_v, pos_v, _qkey, 10, 1023, 48, 64)

        def embe(i, _):
            sl = pl.ds(i * 16, 16)
            i2 = pos_v[pl.ds(i * 16 + 2048, 16)]
            ind = jnp.clip(i2 - 1, 0, G3 - 2)
            c0 = plsc.load_gather(csrt_v, [ind])
            c1 = plsc.load_gather(csrt_v, [ind + 1])
            rq = refg_v[sl]
            t = rq - alpha
            y0 = -1.0 + H_GRID * ind.astype(f32)
            emb_v[sl] = y0 + H_GRID * (t - c0) / (c1 - c0) - rq
            return 0
        lax.fori_loop(0, 16, embe, 0, unroll=4)
        pltpu.sync_copy(emb_v, out_hbm.at[r])
        return 0

    lax.fori_loop(0, 64, do_row, 0)


_sc_call = functools.partial(
    pl.kernel,
    out_type=jax.ShapeDtypeStruct((2048, REF_SIZE), f32),
    mesh=plsc.VectorSubcoreMesh(core_axis_name="c", subcore_axis_name="s"),
    compiler_params=pltpu.CompilerParams(needs_layout_passes=False),
    scratch_types=[
        pltpu.VMEM((G3,), f32),
        pltpu.VMEM((G3,), f32),
        pltpu.VMEM((REF_SIZE,), f32),
        pltpu.VMEM((N,), f32),
        pltpu.VMEM((N,), f32),
        pltpu.VMEM((N,), f32),
        pltpu.VMEM((G3,), f32),
        pltpu.VMEM((G3,), f32),
        pltpu.VMEM((G3,), f32),
        pltpu.VMEM((REF_SIZE,), f32),
        pltpu.VMEM((2048 + 272,), i32),
    ],
)(_sc_embed)


def _loss_body(e1_ref, e2_ref, out_ref):
    d = jnp.abs(e2_ref[...] - e1_ref[...])
    m = jnp.minimum(d, 1.0 - d)
    s = jnp.sum(m * m, axis=2)
    loss = jnp.sqrt(s)
    out_ref[...] = jnp.broadcast_to(jnp.mean(loss, axis=1, keepdims=True), (8, 128))


def kernel(x1, x2, Z):
    Z0 = Z[:, :, 0]
    Z1 = Z[:, :, 1]

    q0, q1 = pl.pallas_call(
        _qr_body,
        out_shape=(jax.ShapeDtypeStruct((NUM_PROJ, D), f32),
                   jax.ShapeDtypeStruct((NUM_PROJ, D), f32)),
    )(Z0, Z1)

    X = jnp.stack([x1, x2]).reshape(16, 1024, D)
    ang = pl.pallas_call(
        _angles_body,
        out_shape=jax.ShapeDtypeStruct((16, NUM_PROJ, N), f32),
        grid=(16,),
        in_specs=[
            pl.BlockSpec((1, 1024, D), lambda i: (i, 0, 0)),
            pl.BlockSpec((NUM_PROJ, D), lambda i: (0, 0)),
            pl.BlockSpec((NUM_PROJ, D), lambda i: (0, 0)),
        ],
        out_specs=pl.BlockSpec((1, NUM_PROJ, N), lambda i: (i, 0, 0)),
    )(X, q0, q1)

    rows = ang.reshape(2048, N)

    xnew = jnp.linspace(-1.0, 2.0, G3).astype(f32)
    int_x = jnp.floor(xnew)
    rest_x = xnew - int_x
    perm = jnp.argsort(rest_x, stable=True)
    tsort = rest_x[perm]
    isort = int_x[perm]
    refg = jnp.linspace(0.0, 1.0, REF_SIZE + 1)[:-1].astype(f32)

    emb = _sc_call(rows, tsort, isort, refg)

    e = emb.reshape(2, 8, NUM_PROJ, REF_SIZE)
    out = pl.pallas_call(
        _loss_body,
        out_shape=jax.ShapeDtypeStruct((8, 128), f32),
    )(e[0], e[1])
    return out[:, 0]


# speedup vs baseline: 35.8178x; 1.8195x over previous
import functools

import jax
import jax.numpy as jnp
import numpy as np
from jax import lax
from jax.experimental import pallas as pl
from jax.experimental.pallas import tpu as pltpu
from jax.experimental.pallas import tpu_sc as plsc

NUM_PROJ = 128
REF_SIZE = 256
D = 128
N = 1024
G3 = 768
INV_N = np.float32(1.0 / 1024.0)
H_GRID = np.float32(3.0 / 767.0)

f32 = jnp.float32
i32 = jnp.int32


def _qr_body(z0_ref, z1_ref, q0_ref, q1_ref):
    z0 = z0_ref[...]
    z1 = z1_ref[...]
    n0sq = jnp.sum(z0 * z0, axis=1, keepdims=True)
    n0 = jnp.sqrt(n0sq)
    z00 = z0[:, 0:1]
    beta = jnp.where(z00 >= 0.0, -n0, n0)
    q0 = z0 / beta
    c = jnp.sum(q0 * z1, axis=1, keepdims=True)
    r = z1 - c * q0
    rn = jnp.sqrt(jnp.sum(r * r, axis=1, keepdims=True))
    udz = jnp.sum(z0 * z1, axis=1, keepdims=True) - beta * z1[:, 0:1]
    uu = 2.0 * n0sq - 2.0 * beta * z00
    w1 = z1[:, 1:2] - 2.0 * (udz / uu) * z0[:, 1:2]
    s1 = jnp.where(w1 >= 0.0, -1.0, 1.0)
    q0_ref[...] = q0
    q1_ref[...] = s1 * (r / rn)


def _angles_body(x_ref, q0_ref, q1_ref, ang_ref):
    x = x_ref[0]
    nrm = jnp.sqrt(jnp.sum(x * x, axis=1, keepdims=True))
    xn = x / jnp.maximum(nrm, 1e-12)
    q0 = q0_ref[...]
    q1 = q1_ref[...]
    dn = (((1,), (1,)), ((), ()))
    p0 = lax.dot_general(q0, xn, dn, preferred_element_type=f32)
    p1 = lax.dot_general(q1, xn, dn, preferred_element_type=f32)
    pn = jnp.maximum(jnp.sqrt(p0 * p0 + p1 * p1), 1e-12)
    pn0 = p0 / pn
    pn1 = p1 / pn
    denom = -pn0
    denom = jnp.where(jnp.abs(denom) < 1e-10, 1e-10, denom)
    ang = (jnp.arctan2(-pn1, denom) + np.float32(np.pi)) / np.float32(2.0 * np.pi)
    ang_ref[0] = ang


def _ones_i32():
    return jnp.ones((16,), i32)


def _floor_i32(x):
    return ((x + 16384.0).astype(i32)) - 16384


def _sc_embed(rows_hbm, tsort_hbm, isort_hbm, refg_hbm, perm_hbm, out_hbm,
              tsort_v, isort_v, refg_v, perm_v,
              row_v, tmp_v, srt_v, sm_v, kbuf_v,
              csrt_v, emb_v, pos_v):
    wid = lax.axis_index("s") * 2 + lax.axis_index("c")
    pltpu.sync_copy(tsort_hbm, tsort_v)
    pltpu.sync_copy(isort_hbm, isort_v)
    pltpu.sync_copy(refg_hbm, refg_v)
    pltpu.sync_copy(perm_hbm, perm_v)

    def do_row(j, _):
        r = wid * 64 + j
        pltpu.sync_copy(rows_hbm.at[r], row_v)

        big = np.float32(3e38)

        def acc_body(i, carry):
            a, m1, m2, mx1, mx2 = carry
            v = row_v[pl.ds(i * 16, 16)]
            nm1 = jnp.minimum(m1, v)
            nm2 = jnp.minimum(m2, jnp.maximum(m1, v))
            nx1 = jnp.maximum(mx1, v)
            nx2 = jnp.maximum(mx2, jnp.minimum(mx1, v))
            return a + v, nm1, nm2, nx1, nx2
        accv, m1v, m2v, x1v, x2v = lax.fori_loop(
            0, 64, acc_body,
            (jnp.zeros((16,), f32), jnp.full((16,), big, f32),
             jnp.full((16,), big, f32), jnp.full((16,), -big, f32),
             jnp.full((16,), -big, f32)), unroll=4)
        alpha = jnp.sum(accv) * np.float32(1.0 / 1024.0) - 0.5
        vmin = jnp.min(m1v)
        vmin2 = jnp.where(jnp.sum((m1v == vmin).astype(i32)) >= 2, vmin,
                          jnp.min(jnp.where(m1v == vmin, m2v, m1v)))
        vmax = jnp.max(x1v)
        vmax2 = jnp.where(jnp.sum((x1v == vmax).astype(i32)) >= 2, vmax,
                          jnp.max(jnp.where(x1v == vmax, x2v, x1v)))

        def zero_t(i, _):
            pos_v[pl.ds(i * 16, 16)] = jnp.zeros((16,), i32)
            return 0
        lax.fori_loop(0, 49, zero_t, 0, unroll=4)

        def tcnt(i, _):
            sl = pl.ds(i * 16, 16)
            v = row_v[sl]
            k = jnp.clip(_floor_i32(v * 767.0) + 2, 2, G3)
            t0 = plsc.load_gather(tsort_v, [k - 1])
            t1 = plsc.load_gather(tsort_v, [jnp.minimum(k, G3 - 1)])
            k = (k
                 - ((k >= 3) & (t0 > v)).astype(i32)
                 + ((k <= G3 - 1) & (t1 <= v)).astype(i32))
            kbuf_v[sl] = k
            plsc.addupdate_scatter(pos_v, [k], _ones_i32())
            return 0
        lax.fori_loop(0, 64, tcnt, 0, unroll=4)

        def cum_t(i, c):
            h = pos_v[pl.ds(i * 16, 16)]
            cs = plsc.cumsum(h)
            pos_v[pl.ds(i * 16, 16)] = cs - h + c
            return c + jnp.sum(h)
        lax.fori_loop(0, 49, cum_t, jnp.int32(0), unroll=2)

        def place_g(i, _):
            sl = pl.ds(i * 16, 16)
            k = kbuf_v[sl]
            b = plsc.load_gather(pos_v, [k])
            cnt, _unused = plsc.scan_count(k)
            plsc.store_scatter(tmp_v, [b + cnt.astype(i32) - 1], row_v[sl])
            plsc.addupdate_scatter(pos_v, [k], _ones_i32())
            return 0
        lax.fori_loop(0, 64, place_g, 0, unroll=2)

        def pmax(i, c):
            sl = pl.ds(i * 16, 16)
            g = tmp_v[sl]
            srt_v[sl] = jnp.maximum(plsc.cummax(g), c)
            return jnp.maximum(c, jnp.max(g))
        lax.fori_loop(0, 64, pmax, -big, unroll=2)

        def smin(i, c):
            sl = pl.ds((63 - i) * 16, 16)
            g = tmp_v[sl]
            rcm = -plsc.cummax(-lax.rev(g, (0,)))
            sm_v[sl] = jnp.minimum(lax.rev(rcm, (0,)), c)
            return jnp.minimum(c, jnp.min(g))
        lax.fori_loop(0, 64, smin, big, unroll=2)

        def cdfe(i, _):
            sl = pl.ds(i * 16, 16)
            iq = pos_v[sl]
            lo = iq == 0
            hi = iq == N
            x0 = plsc.load_gather(srt_v, [jnp.clip(iq - 1, 0, N - 1)])
            x1 = plsc.load_gather(sm_v, [jnp.clip(iq, 0, N - 1)])
            x0 = jnp.where(lo, vmin, jnp.where(hi, vmax2, x0))
            x1 = jnp.where(lo, vmin2, jnp.where(hi, vmax, x1))
            y0 = jnp.clip(iq, 1, N - 1).astype(f32) * INV_N
            tq = tsort_v[sl]
            val = isort_v[sl] + y0 + INV_N * (tq - x0) / (x1 - x0)
            plsc.store_scatter(csrt_v, [perm_v[sl]], val)
            return 0
        lax.fori_loop(0, 48, cdfe, 0, unroll=4)

        def zero_q(i, _):
            pos_v[pl.ds(i * 16 + 2048, 16)] = jnp.zeros((16,), i32)
            return 0
        lax.fori_loop(0, 17, zero_q, 0, unroll=4)

        def qcnt(i, _):
            c = csrt_v[pl.ds(i * 16, 16)]
            p = _floor_i32((c + alpha) * 256.0)
            p = jnp.clip(p + 1, 0, 256)
            r0 = plsc.load_gather(refg_v, [jnp.clip(p - 1, 0, 255)])
            r1 = plsc.load_gather(refg_v, [jnp.clip(p, 0, 255)])
            p = (p
                 - ((p >= 1) & (r0 - alpha > c)).astype(i32)
                 + ((p <= 255) & (r1 - alpha <= c)).astype(i32))
            plsc.addupdate_scatter(pos_v, [p + 2048], _ones_i32())
            return 0
        lax.fori_loop(0, 48, qcnt, 0, unroll=4)

        def cum_q(i, c):
            h = pos_v[pl.ds(i * 16 + 2048, 16)]
            cs = plsc.cumsum(h) + c
            pos_v[pl.ds(i * 16 + 2048, 16)] = cs
            return c + jnp.sum(h)
        lax.fori_loop(0, 16, cum_q, jnp.int32(0), unroll=2)

        def embe(i, _):
            sl = pl.ds(i * 16, 16)
            i2 = pos_v[pl.ds(i * 16 + 2048, 16)]
            ind = jnp.clip(i2 - 1, 0, G3 - 2)
            c0 = plsc.load_gather(csrt_v, [ind])
            c1 = plsc.load_gather(csrt_v, [ind + 1])
            rq = refg_v[sl]
            t = rq - alpha
            y0 = -1.0 + H_GRID * ind.astype(f32)
            emb_v[sl] = y0 + H_GRID * (t - c0) / (c1 - c0) - rq
            return 0
        lax.fori_loop(0, 16, embe, 0, unroll=4)
        pltpu.sync_copy(emb_v, out_hbm.at[r])
        return 0

    lax.fori_loop(0, 64, do_row, 0)


_sc_call = functools.partial(
    pl.kernel,
    out_type=jax.ShapeDtypeStruct((2048, REF_SIZE), f32),
    mesh=plsc.VectorSubcoreMesh(core_axis_name="c", subcore_axis_name="s"),
    compiler_params=pltpu.CompilerParams(needs_layout_passes=False),
    scratch_types=[
        pltpu.VMEM((G3,), f32),
        pltpu.VMEM((G3,), f32),
        pltpu.VMEM((REF_SIZE,), f32),
        pltpu.VMEM((G3,), i32),
        pltpu.VMEM((N,), f32),
        pltpu.VMEM((N,), f32),
        pltpu.VMEM((N,), f32),
        pltpu.VMEM((N,), f32),
        pltpu.VMEM((N,), i32),
        pltpu.VMEM((G3,), f32),
        pltpu.VMEM((REF_SIZE,), f32),
        pltpu.VMEM((2048 + 272,), i32),
    ],
)(_sc_embed)


def _loss_body(e1_ref, e2_ref, out_ref):
    d = jnp.abs(e2_ref[...] - e1_ref[...])
    m = jnp.minimum(d, 1.0 - d)
    s = jnp.sum(m * m, axis=2)
    loss = jnp.sqrt(s)
    out_ref[...] = jnp.broadcast_to(jnp.mean(loss, axis=1, keepdims=True), (8, 128))


def kernel(x1, x2, Z):
    Z0 = Z[:, :, 0]
    Z1 = Z[:, :, 1]

    q0, q1 = pl.pallas_call(
        _qr_body,
        out_shape=(jax.ShapeDtypeStruct((NUM_PROJ, D), f32),
                   jax.ShapeDtypeStruct((NUM_PROJ, D), f32)),
    )(Z0, Z1)

    X = jnp.stack([x1, x2]).reshape(16, 1024, D)
    ang = pl.pallas_call(
        _angles_body,
        out_shape=jax.ShapeDtypeStruct((16, NUM_PROJ, N), f32),
        grid=(16,),
        in_specs=[
            pl.BlockSpec((1, 1024, D), lambda i: (i, 0, 0)),
            pl.BlockSpec((NUM_PROJ, D), lambda i: (0, 0)),
            pl.BlockSpec((NUM_PROJ, D), lambda i: (0, 0)),
        ],
        out_specs=pl.BlockSpec((1, NUM_PROJ, N), lambda i: (i, 0, 0)),
    )(X, q0, q1)

    rows = ang.reshape(2048, N)

    xnew = jnp.linspace(-1.0, 2.0, G3).astype(f32)
    int_x = jnp.floor(xnew)
    rest_x = xnew - int_x
    perm = jnp.argsort(rest_x, stable=True)
    tsort = rest_x[perm]
    isort = int_x[perm]
    refg = jnp.linspace(0.0, 1.0, REF_SIZE + 1)[:-1].astype(f32)

    emb = _sc_call(rows, tsort, isort, refg, perm.astype(i32))

    e = emb.reshape(2, 8, NUM_PROJ, REF_SIZE)
    out = pl.pallas_call(
        _loss_body,
        out_shape=jax.ShapeDtypeStruct((8, 128), f32),
    )(e[0], e[1])
    return out[:, 0]


# speedup vs baseline: 37.2881x; 1.0411x over previous
import functools

import jax
import jax.numpy as jnp
import numpy as np
from jax import lax
from jax.experimental import pallas as pl
from jax.experimental.pallas import tpu as pltpu
from jax.experimental.pallas import tpu_sc as plsc

NUM_PROJ = 128
REF_SIZE = 256
D = 128
N = 1024
G3 = 768
INV_N = np.float32(1.0 / 1024.0)
H_GRID = np.float32(3.0 / 767.0)

f32 = jnp.float32
i32 = jnp.int32


def _qr_body(z0_ref, z1_ref, q0_ref, q1_ref):
    z0 = z0_ref[...]
    z1 = z1_ref[...]
    n0sq = jnp.sum(z0 * z0, axis=1, keepdims=True)
    n0 = jnp.sqrt(n0sq)
    z00 = z0[:, 0:1]
    beta = jnp.where(z00 >= 0.0, -n0, n0)
    q0 = z0 / beta
    c = jnp.sum(q0 * z1, axis=1, keepdims=True)
    r = z1 - c * q0
    rn = jnp.sqrt(jnp.sum(r * r, axis=1, keepdims=True))
    udz = jnp.sum(z0 * z1, axis=1, keepdims=True) - beta * z1[:, 0:1]
    uu = 2.0 * n0sq - 2.0 * beta * z00
    w1 = z1[:, 1:2] - 2.0 * (udz / uu) * z0[:, 1:2]
    s1 = jnp.where(w1 >= 0.0, -1.0, 1.0)
    q0_ref[...] = q0
    q1_ref[...] = s1 * (r / rn)


def _angles_body(x_ref, q0_ref, q1_ref, ang_ref):
    x = x_ref[0]
    nrm = jnp.sqrt(jnp.sum(x * x, axis=1, keepdims=True))
    xn = x / jnp.maximum(nrm, 1e-12)
    q0 = q0_ref[...]
    q1 = q1_ref[...]
    dn = (((1,), (1,)), ((), ()))
    p0 = lax.dot_general(q0, xn, dn, preferred_element_type=f32)
    p1 = lax.dot_general(q1, xn, dn, preferred_element_type=f32)
    pn = jnp.maximum(jnp.sqrt(p0 * p0 + p1 * p1), 1e-12)
    pn0 = p0 / pn
    pn1 = p1 / pn
    denom = -pn0
    denom = jnp.where(jnp.abs(denom) < 1e-10, 1e-10, denom)
    ang = (jnp.arctan2(-pn1, denom) + np.float32(np.pi)) / np.float32(2.0 * np.pi)
    ang_ref[0] = ang


def _ones_i32():
    return jnp.ones((16,), i32)


def _floor_i32(x):
    return ((x + 16384.0).astype(i32)) - 16384


def _sc_embed(rows_hbm, tsort_hbm, isort_hbm, refg_hbm, perm_hbm, out_hbm,
              tsort_v, isort_v, refg_v, perm_v,
              row_v, tmp_v, srt_v, sm_v, kbuf_v,
              csrt_v, emb_v, pos_v):
    wid = lax.axis_index("s") * 2 + lax.axis_index("c")
    pltpu.sync_copy(tsort_hbm, tsort_v)
    pltpu.sync_copy(isort_hbm, isort_v)
    pltpu.sync_copy(refg_hbm, refg_v)
    pltpu.sync_copy(perm_hbm, perm_v)

    def do_row(j, _):
        r = wid * 64 + j
        pltpu.sync_copy(rows_hbm.at[r], row_v)

        big = np.float32(3e38)

        def zero_t(i, _):
            pos_v[pl.ds(i * 16, 16)] = jnp.zeros((16,), i32)
            return 0
        lax.fori_loop(0, 49, zero_t, 0, unroll=4)

        def zero_q(i, _):
            pos_v[pl.ds(i * 16 + 2048, 16)] = jnp.zeros((16,), i32)
            return 0
        lax.fori_loop(0, 17, zero_q, 0, unroll=4)

        def tcnt(i, carry):
            a, m1, m2, mx1, mx2 = carry
            sl = pl.ds(i * 16, 16)
            v = row_v[sl]
            k = jnp.clip(_floor_i32(v * 767.0) + 2, 2, G3)
            t0 = plsc.load_gather(tsort_v, [k - 1])
            t1 = plsc.load_gather(tsort_v, [jnp.minimum(k, G3 - 1)])
            k = (k
                 - ((k >= 3) & (t0 > v)).astype(i32)
                 + ((k <= G3 - 1) & (t1 <= v)).astype(i32))
            kbuf_v[sl] = k
            plsc.addupdate_scatter(pos_v, [k], _ones_i32())
            nm1 = jnp.minimum(m1, v)
            nm2 = jnp.minimum(m2, jnp.maximum(m1, v))
            nx1 = jnp.maximum(mx1, v)
            nx2 = jnp.maximum(mx2, jnp.minimum(mx1, v))
            return a + v, nm1, nm2, nx1, nx2
        accv, m1v, m2v, x1v, x2v = lax.fori_loop(
            0, 64, tcnt,
            (jnp.zeros((16,), f32), jnp.full((16,), big, f32),
             jnp.full((16,), big, f32), jnp.full((16,), -big, f32),
             jnp.full((16,), -big, f32)), unroll=4)
        alpha = jnp.sum(accv) * np.float32(1.0 / 1024.0) - 0.5
        vmin = jnp.min(m1v)
        vmin2 = jnp.where(jnp.sum((m1v == vmin).astype(i32)) >= 2, vmin,
                          jnp.min(jnp.where(m1v == vmin, m2v, m1v)))
        vmax = jnp.max(x1v)
        vmax2 = jnp.where(jnp.sum((x1v == vmax).astype(i32)) >= 2, vmax,
                          jnp.max(jnp.where(x1v == vmax, x2v, x1v)))

        def cum_t(i, c):
            h = pos_v[pl.ds(i * 16, 16)]
            cs = plsc.cumsum(h)
            pos_v[pl.ds(i * 16, 16)] = cs - h + c
            return c + jnp.sum(h)
        lax.fori_loop(0, 49, cum_t, jnp.int32(0), unroll=2)

        def place_g(i, _):
            sl = pl.ds(i * 16, 16)
            k = kbuf_v[sl]
            b = plsc.load_gather(pos_v, [k])
            cnt, _unused = plsc.scan_count(k)
            plsc.store_scatter(tmp_v, [b + cnt.astype(i32) - 1], row_v[sl])
            plsc.addupdate_scatter(pos_v, [k], _ones_i32())
            return 0
        lax.fori_loop(0, 64, place_g, 0, unroll=2)

        def pmsm(i, carry):
            cx, cn = carry
            slf = pl.ds(i * 16, 16)
            gf = tmp_v[slf]
            srt_v[slf] = jnp.maximum(plsc.cummax(gf), cx)
            slr = pl.ds((63 - i) * 16, 16)
            gr = tmp_v[slr]
            rcm = -plsc.cummax(-lax.rev(gr, (0,)))
            sm_v[slr] = jnp.minimum(lax.rev(rcm, (0,)), cn)
            return jnp.maximum(cx, jnp.max(gf)), jnp.minimum(cn, jnp.min(gr))
        lax.fori_loop(0, 64, pmsm, (-big, big), unroll=2)

        def cdfe(i, _):
            sl = pl.ds(i * 16, 16)
            iq = pos_v[sl]
            lo = iq == 0
            hi = iq == N
            x0 = plsc.load_gather(srt_v, [jnp.clip(iq - 1, 0, N - 1)])
            x1 = plsc.load_gather(sm_v, [jnp.clip(iq, 0, N - 1)])
            x0 = jnp.where(lo, vmin, jnp.where(hi, vmax2, x0))
            x1 = jnp.where(lo, vmin2, jnp.where(hi, vmax, x1))
            y0 = jnp.clip(iq, 1, N - 1).astype(f32) * INV_N
            tq = tsort_v[sl]
            c = isort_v[sl] + y0 + INV_N * (tq - x0) / (x1 - x0)
            plsc.store_scatter(csrt_v, [perm_v[sl]], c)
            p = _floor_i32((c + alpha) * 256.0)
            p = jnp.clip(p + 1, 0, 256)
            r0 = plsc.load_gather(refg_v, [jnp.clip(p - 1, 0, 255)])
            r1 = plsc.load_gather(refg_v, [jnp.clip(p, 0, 255)])
            p = (p
                 - ((p >= 1) & (r0 - alpha > c)).astype(i32)
                 + ((p <= 255) & (r1 - alpha <= c)).astype(i32))
            plsc.addupdate_scatter(pos_v, [p + 2048], _ones_i32())
            return 0
        lax.fori_loop(0, 48, cdfe, 0, unroll=4)

        def cum_q(i, c):
            h = pos_v[pl.ds(i * 16 + 2048, 16)]
            cs = plsc.cumsum(h) + c
            pos_v[pl.ds(i * 16 + 2048, 16)] = cs
            return c + jnp.sum(h)
        lax.fori_loop(0, 16, cum_q, jnp.int32(0), unroll=2)

        def embe(i, _):
            sl = pl.ds(i * 16, 16)
            i2 = pos_v[pl.ds(i * 16 + 2048, 16)]
            ind = jnp.clip(i2 - 1, 0, G3 - 2)
            c0 = plsc.load_gather(csrt_v, [ind])
            c1 = plsc.load_gather(csrt_v, [ind + 1])
            rq = refg_v[sl]
            t = rq - alpha
            y0 = -1.0 + H_GRID * ind.astype(f32)
            emb_v[sl] = y0 + H_GRID * (t - c0) / (c1 - c0) - rq
            return 0
        lax.fori_loop(0, 16, embe, 0, unroll=4)
        pltpu.sync_copy(emb_v, out_hbm.at[r])
        return 0

    lax.fori_loop(0, 64, do_row, 0)


_sc_call = functools.partial(
    pl.kernel,
    out_type=jax.ShapeDtypeStruct((2048, REF_SIZE), f32),
    mesh=plsc.VectorSubcoreMesh(core_axis_name="c", subcore_axis_name="s"),
    compiler_params=pltpu.CompilerParams(needs_layout_passes=False),
    scratch_types=[
        pltpu.VMEM((G3,), f32),
        pltpu.VMEM((G3,), f32),
        pltpu.VMEM((REF_SIZE,), f32),
        pltpu.VMEM((G3,), i32),
        pltpu.VMEM((N,), f32),
        pltpu.VMEM((N,), f32),
        pltpu.VMEM((N,), f32),
        pltpu.VMEM((N,), f32),
        pltpu.VMEM((N,), i32),
        pltpu.VMEM((G3,), f32),
        pltpu.VMEM((REF_SIZE,), f32),
        pltpu.VMEM((2048 + 272,), i32),
    ],
)(_sc_embed)


def _loss_body(e1_ref, e2_ref, out_ref):
    d = jnp.abs(e2_ref[...] - e1_ref[...])
    m = jnp.minimum(d, 1.0 - d)
    s = jnp.sum(m * m, axis=2)
    loss = jnp.sqrt(s)
    out_ref[...] = jnp.broadcast_to(jnp.mean(loss, axis=1, keepdims=True), (8, 128))


def kernel(x1, x2, Z):
    Z0 = Z[:, :, 0]
    Z1 = Z[:, :, 1]

    q0, q1 = pl.pallas_call(
        _qr_body,
        out_shape=(jax.ShapeDtypeStruct((NUM_PROJ, D), f32),
                   jax.ShapeDtypeStruct((NUM_PROJ, D), f32)),
    )(Z0, Z1)

    X = jnp.stack([x1, x2]).reshape(16, 1024, D)
    ang = pl.pallas_call(
        _angles_body,
        out_shape=jax.ShapeDtypeStruct((16, NUM_PROJ, N), f32),
        grid=(16,),
        in_specs=[
            pl.BlockSpec((1, 1024, D), lambda i: (i, 0, 0)),
            pl.BlockSpec((NUM_PROJ, D), lambda i: (0, 0)),
            pl.BlockSpec((NUM_PROJ, D), lambda i: (0, 0)),
        ],
        out_specs=pl.BlockSpec((1, NUM_PROJ, N), lambda i: (i, 0, 0)),
    )(X, q0, q1)

    rows = ang.reshape(2048, N)

    xnew = jnp.linspace(-1.0, 2.0, G3).astype(f32)
    int_x = jnp.floor(xnew)
    rest_x = xnew - int_x
    perm = jnp.argsort(rest_x, stable=True)
    tsort = rest_x[perm]
    isort = int_x[perm]
    refg = jnp.linspace(0.0, 1.0, REF_SIZE + 1)[:-1].astype(f32)

    emb = _sc_call(rows, tsort, isort, refg, perm.astype(i32))

    e = emb.reshape(2, 8, NUM_PROJ, REF_SIZE)
    out = pl.pallas_call(
        _loss_body,
        out_shape=jax.ShapeDtypeStruct((8, 128), f32),
    )(e[0], e[1])
    return out[:, 0]


# speedup vs baseline: 39.7011x; 1.0647x over previous
import functools

import jax
import jax.numpy as jnp
import numpy as np
from jax import lax
from jax.experimental import pallas as pl
from jax.experimental.pallas import tpu as pltpu
from jax.experimental.pallas import tpu_sc as plsc

NUM_PROJ = 128
REF_SIZE = 256
D = 128
N = 1024
G3 = 768
INV_N = np.float32(1.0 / 1024.0)
H_GRID = np.float32(3.0 / 767.0)

f32 = jnp.float32
i32 = jnp.int32


def _qr_body(z0_ref, z1_ref, q0_ref, q1_ref):
    z0 = z0_ref[...]
    z1 = z1_ref[...]
    n0sq = jnp.sum(z0 * z0, axis=1, keepdims=True)
    n0 = jnp.sqrt(n0sq)
    z00 = z0[:, 0:1]
    beta = jnp.where(z00 >= 0.0, -n0, n0)
    q0 = z0 / beta
    c = jnp.sum(q0 * z1, axis=1, keepdims=True)
    r = z1 - c * q0
    rn = jnp.sqrt(jnp.sum(r * r, axis=1, keepdims=True))
    udz = jnp.sum(z0 * z1, axis=1, keepdims=True) - beta * z1[:, 0:1]
    uu = 2.0 * n0sq - 2.0 * beta * z00
    w1 = z1[:, 1:2] - 2.0 * (udz / uu) * z0[:, 1:2]
    s1 = jnp.where(w1 >= 0.0, -1.0, 1.0)
    q0_ref[...] = q0
    q1_ref[...] = s1 * (r / rn)


def _angles_body(x_ref, q0_ref, q1_ref, ang_ref):
    x = x_ref[0]
    nrm = jnp.sqrt(jnp.sum(x * x, axis=1, keepdims=True))
    xn = x / jnp.maximum(nrm, 1e-12)
    q0 = q0_ref[...]
    q1 = q1_ref[...]
    dn = (((1,), (1,)), ((), ()))
    p0 = lax.dot_general(q0, xn, dn, preferred_element_type=f32)
    p1 = lax.dot_general(q1, xn, dn, preferred_element_type=f32)
    pn = jnp.maximum(jnp.sqrt(p0 * p0 + p1 * p1), 1e-12)
    pn0 = p0 / pn
    pn1 = p1 / pn
    denom = -pn0
    denom = jnp.where(jnp.abs(denom) < 1e-10, 1e-10, denom)
    ang = (jnp.arctan2(-pn1, denom) + np.float32(np.pi)) / np.float32(2.0 * np.pi)
    ang_ref[0] = ang


def _ones_i32():
    return jnp.ones((16,), i32)


def _floor_i32(x):
    return ((x + 16384.0).astype(i32)) - 16384


def _sc_embed(rows_hbm, tsort_hbm, isort_hbm, refg_hbm, perm_hbm, out_hbm,
              tsort_v, isort_v, refg_v, perm_v,
              rowb_v, tmp_v, srt_v, sm_v, kbuf_v,
              csrt_v, embb_v, pos_v):
    wid = lax.axis_index("s") * 2 + lax.axis_index("c")
    pltpu.sync_copy(tsort_hbm, tsort_v)
    pltpu.sync_copy(isort_hbm, isort_v)
    pltpu.sync_copy(refg_hbm, refg_v)
    pltpu.sync_copy(perm_hbm, perm_v)
    pltpu.sync_copy(rows_hbm.at[pl.ds(wid * 64, 64)], rowb_v)

    def do_row(j, _):
        big = np.float32(3e38)

        def zero_t(i, _):
            pos_v[pl.ds(i * 16, 16)] = jnp.zeros((16,), i32)
            return 0
        lax.fori_loop(0, 49, zero_t, 0, unroll=4)

        def zero_q(i, _):
            pos_v[pl.ds(i * 16 + 2048, 16)] = jnp.zeros((16,), i32)
            return 0
        lax.fori_loop(0, 17, zero_q, 0, unroll=4)

        def tcnt(i, carry):
            a, m1, m2, mx1, mx2 = carry
            sl = pl.ds(i * 16, 16)
            v = rowb_v[j, sl]
            k = jnp.clip(_floor_i32(v * 767.0) + 2, 2, G3)
            t0 = plsc.load_gather(tsort_v, [k - 1])
            t1 = plsc.load_gather(tsort_v, [jnp.minimum(k, G3 - 1)])
            k = (k
                 - ((k >= 3) & (t0 > v)).astype(i32)
                 + ((k <= G3 - 1) & (t1 <= v)).astype(i32))
            kbuf_v[sl] = k
            plsc.addupdate_scatter(pos_v, [k], _ones_i32())
            nm1 = jnp.minimum(m1, v)
            nm2 = jnp.minimum(m2, jnp.maximum(m1, v))
            nx1 = jnp.maximum(mx1, v)
            nx2 = jnp.maximum(mx2, jnp.minimum(mx1, v))
            return a + v, nm1, nm2, nx1, nx2
        accv, m1v, m2v, x1v, x2v = lax.fori_loop(
            0, 64, tcnt,
            (jnp.zeros((16,), f32), jnp.full((16,), big, f32),
             jnp.full((16,), big, f32), jnp.full((16,), -big, f32),
             jnp.full((16,), -big, f32)), unroll=4)
        alpha = jnp.sum(accv) * np.float32(1.0 / 1024.0) - 0.5
        vmin = jnp.min(m1v)
        vmin2 = jnp.where(jnp.sum((m1v == vmin).astype(i32)) >= 2, vmin,
                          jnp.min(jnp.where(m1v == vmin, m2v, m1v)))
        vmax = jnp.max(x1v)
        vmax2 = jnp.where(jnp.sum((x1v == vmax).astype(i32)) >= 2, vmax,
                          jnp.max(jnp.where(x1v == vmax, x2v, x1v)))

        def cum_t(i, c):
            h = pos_v[pl.ds(i * 16, 16)]
            cs = plsc.cumsum(h)
            pos_v[pl.ds(i * 16, 16)] = cs - h + c
            return c + jnp.sum(h)
        lax.fori_loop(0, 49, cum_t, jnp.int32(0), unroll=2)

        def place_g(i, _):
            sl = pl.ds(i * 16, 16)
            k = kbuf_v[sl]
            b = plsc.load_gather(pos_v, [k])
            cnt, _unused = plsc.scan_count(k)
            plsc.store_scatter(tmp_v, [b + cnt.astype(i32) - 1], rowb_v[j, sl])
            plsc.addupdate_scatter(pos_v, [k], _ones_i32())
            return 0
        lax.fori_loop(0, 64, place_g, 0, unroll=2)

        def pmsm(i, carry):
            cx, cn = carry
            slf = pl.ds(i * 16, 16)
            gf = tmp_v[slf]
            srt_v[slf] = jnp.maximum(plsc.cummax(gf), cx)
            slr = pl.ds((63 - i) * 16, 16)
            gr = tmp_v[slr]
            rcm = -plsc.cummax(-lax.rev(gr, (0,)))
            sm_v[slr] = jnp.minimum(lax.rev(rcm, (0,)), cn)
            return jnp.maximum(cx, jnp.max(gf)), jnp.minimum(cn, jnp.min(gr))
        lax.fori_loop(0, 64, pmsm, (-big, big), unroll=2)

        def cdfe(i, _):
            sl = pl.ds(i * 16, 16)
            iq = pos_v[sl]
            lo = iq == 0
            hi = iq == N
            x0 = plsc.load_gather(srt_v, [jnp.clip(iq - 1, 0, N - 1)])
            x1 = plsc.load_gather(sm_v, [jnp.clip(iq, 0, N - 1)])
            x0 = jnp.where(lo, vmin, jnp.where(hi, vmax2, x0))
            x1 = jnp.where(lo, vmin2, jnp.where(hi, vmax, x1))
            y0 = jnp.clip(iq, 1, N - 1).astype(f32) * INV_N
            tq = tsort_v[sl]
            c = isort_v[sl] + y0 + INV_N * (tq - x0) / (x1 - x0)
            plsc.store_scatter(csrt_v, [perm_v[sl]], c)
            p = _floor_i32((c + alpha) * 256.0)
            p = jnp.clip(p + 1, 0, 256)
            r0 = plsc.load_gather(refg_v, [jnp.clip(p - 1, 0, 255)])
            r1 = plsc.load_gather(refg_v, [jnp.clip(p, 0, 255)])
            p = (p
                 - ((p >= 1) & (r0 - alpha > c)).astype(i32)
                 + ((p <= 255) & (r1 - alpha <= c)).astype(i32))
            plsc.addupdate_scatter(pos_v, [p + 2048], _ones_i32())
            return 0
        lax.fori_loop(0, 48, cdfe, 0, unroll=4)

        def cum_q(i, c):
            h = pos_v[pl.ds(i * 16 + 2048, 16)]
            cs = plsc.cumsum(h) + c
            pos_v[pl.ds(i * 16 + 2048, 16)] = cs
            return c + jnp.sum(h)
        lax.fori_loop(0, 16, cum_q, jnp.int32(0), unroll=2)

        def embe(i, _):
            sl = pl.ds(i * 16, 16)
            i2 = pos_v[pl.ds(i * 16 + 2048, 16)]
            ind = jnp.clip(i2 - 1, 0, G3 - 2)
            c0 = plsc.load_gather(csrt_v, [ind])
            c1 = plsc.load_gather(csrt_v, [ind + 1])
            rq = refg_v[sl]
            t = rq - alpha
            y0 = -1.0 + H_GRID * ind.astype(f32)
            embb_v[j, sl] = y0 + H_GRID * (t - c0) / (c1 - c0) - rq
            return 0
        lax.fori_loop(0, 16, embe, 0, unroll=4)
        return 0

    lax.fori_loop(0, 64, do_row, 0)
    pltpu.sync_copy(embb_v, out_hbm.at[pl.ds(wid * 64, 64)])


_sc_call = functools.partial(
    pl.kernel,
    out_type=jax.ShapeDtypeStruct((2048, REF_SIZE), f32),
    mesh=plsc.VectorSubcoreMesh(core_axis_name="c", subcore_axis_name="s"),
    compiler_params=pltpu.CompilerParams(needs_layout_passes=False),
    scratch_types=[
        pltpu.VMEM((G3,), f32),
        pltpu.VMEM((G3,), f32),
        pltpu.VMEM((REF_SIZE,), f32),
        pltpu.VMEM((G3,), i32),
        pltpu.VMEM((64, N), f32),
        pltpu.VMEM((N,), f32),
        pltpu.VMEM((N,), f32),
        pltpu.VMEM((N,), f32),
        pltpu.VMEM((N,), i32),
        pltpu.VMEM((G3,), f32),
        pltpu.VMEM((64, REF_SIZE), f32),
        pltpu.VMEM((2048 + 272,), i32),
    ],
)(_sc_embed)


def _loss_body(e1_ref, e2_ref, out_ref):
    d = jnp.abs(e2_ref[...] - e1_ref[...])
    m = jnp.minimum(d, 1.0 - d)
    s = jnp.sum(m * m, axis=2)
    loss = jnp.sqrt(s)
    out_ref[...] = jnp.broadcast_to(jnp.mean(loss, axis=1, keepdims=True), (8, 128))


def kernel(x1, x2, Z):
    Z0 = Z[:, :, 0]
    Z1 = Z[:, :, 1]

    q0, q1 = pl.pallas_call(
        _qr_body,
        out_shape=(jax.ShapeDtypeStruct((NUM_PROJ, D), f32),
                   jax.ShapeDtypeStruct((NUM_PROJ, D), f32)),
    )(Z0, Z1)

    X = jnp.stack([x1, x2]).reshape(16, 1024, D)
    ang = pl.pallas_call(
        _angles_body,
        out_shape=jax.ShapeDtypeStruct((16, NUM_PROJ, N), f32),
        grid=(16,),
        in_specs=[
            pl.BlockSpec((1, 1024, D), lambda i: (i, 0, 0)),
            pl.BlockSpec((NUM_PROJ, D), lambda i: (0, 0)),
            pl.BlockSpec((NUM_PROJ, D), lambda i: (0, 0)),
        ],
        out_specs=pl.BlockSpec((1, NUM_PROJ, N), lambda i: (i, 0, 0)),
    )(X, q0, q1)

    rows = ang.reshape(2048, N)

    xnew = jnp.linspace(-1.0, 2.0, G3).astype(f32)
    int_x = jnp.floor(xnew)
    rest_x = xnew - int_x
    perm = jnp.argsort(rest_x, stable=True)
    tsort = rest_x[perm]
    isort = int_x[perm]
    refg = jnp.linspace(0.0, 1.0, REF_SIZE + 1)[:-1].astype(f32)

    emb = _sc_call(rows, tsort, isort, refg, perm.astype(i32))

    e = emb.reshape(2, 8, NUM_PROJ, REF_SIZE)
    out = pl.pallas_call(
        _loss_body,
        out_shape=jax.ShapeDtypeStruct((8, 128), f32),
    )(e[0], e[1])
    return out[:, 0]


# speedup vs baseline: 40.1218x; 1.0106x over previous
import functools

import jax
import jax.numpy as jnp
import numpy as np
from jax import lax
from jax.experimental import pallas as pl
from jax.experimental.pallas import tpu as pltpu
from jax.experimental.pallas import tpu_sc as plsc

NUM_PROJ = 128
REF_SIZE = 256
D = 128
N = 1024
G3 = 768
INV_N = np.float32(1.0 / 1024.0)
H_GRID = np.float32(3.0 / 767.0)

f32 = jnp.float32
i32 = jnp.int32


def _qr_body(z0_ref, z1_ref, q0_ref, q1_ref):
    z0 = z0_ref[...]
    z1 = z1_ref[...]
    n0sq = jnp.sum(z0 * z0, axis=1, keepdims=True)
    n0 = jnp.sqrt(n0sq)
    z00 = z0[:, 0:1]
    beta = jnp.where(z00 >= 0.0, -n0, n0)
    q0 = z0 / beta
    c = jnp.sum(q0 * z1, axis=1, keepdims=True)
    r = z1 - c * q0
    rn = jnp.sqrt(jnp.sum(r * r, axis=1, keepdims=True))
    udz = jnp.sum(z0 * z1, axis=1, keepdims=True) - beta * z1[:, 0:1]
    uu = 2.0 * n0sq - 2.0 * beta * z00
    w1 = z1[:, 1:2] - 2.0 * (udz / uu) * z0[:, 1:2]
    s1 = jnp.where(w1 >= 0.0, -1.0, 1.0)
    q0_ref[...] = q0
    q1_ref[...] = s1 * (r / rn)


def _angles_body(x_ref, q0_ref, q1_ref, ang_ref):
    x = x_ref[0]
    nrm = jnp.sqrt(jnp.sum(x * x, axis=1, keepdims=True))
    xn = x / jnp.maximum(nrm, 1e-12)
    q0 = q0_ref[...]
    q1 = q1_ref[...]
    dn = (((1,), (1,)), ((), ()))
    p0 = lax.dot_general(q0, xn, dn, preferred_element_type=f32)
    p1 = lax.dot_general(q1, xn, dn, preferred_element_type=f32)
    pn = jnp.maximum(jnp.sqrt(p0 * p0 + p1 * p1), 1e-12)
    pn0 = p0 / pn
    pn1 = p1 / pn
    denom = -pn0
    denom = jnp.where(jnp.abs(denom) < 1e-10, 1e-10, denom)
    ang = (jnp.arctan2(-pn1, denom) + np.float32(np.pi)) / np.float32(2.0 * np.pi)
    ang_ref[0] = ang


def _ones_i32():
    return jnp.ones((16,), i32)


def _floor_i32(x):
    return ((x + 16384.0).astype(i32)) - 16384


def _sc_embed(rows_hbm, tsort_hbm, isort_hbm, refg_hbm, perm_hbm, out_hbm,
              tsort_v, isort_v, refg_v, perm_v,
              rowb_v, tmp_v, srt_v, sm_v, kbuf_v,
              csrt_v, embb_v, pos_v):
    wid = lax.axis_index("s") * 2 + lax.axis_index("c")
    pltpu.sync_copy(tsort_hbm, tsort_v)
    pltpu.sync_copy(isort_hbm, isort_v)
    pltpu.sync_copy(refg_hbm, refg_v)
    pltpu.sync_copy(perm_hbm, perm_v)
    pltpu.sync_copy(rows_hbm.at[pl.ds(wid * 64, 64)], rowb_v)

    def do_row(j, _):
        big = np.float32(3e38)

        def zero_t(i, _):
            pos_v[pl.ds(i * 16, 16)] = jnp.zeros((16,), i32)
            return 0
        lax.fori_loop(0, 49, zero_t, 0, unroll=4)

        def zero_q(i, _):
            pos_v[pl.ds(i * 16 + 2048, 16)] = jnp.zeros((16,), i32)
            return 0
        lax.fori_loop(0, 17, zero_q, 0, unroll=4)

        def tcnt(i, carry):
            a, m1, m2, mx1, mx2 = carry
            sl = pl.ds(i * 16, 16)
            v = rowb_v[j, sl]
            k = jnp.clip(_floor_i32(v * 767.0) + 2, 2, G3)
            t0 = plsc.load_gather(tsort_v, [k - 1])
            t1 = plsc.load_gather(tsort_v, [jnp.minimum(k, G3 - 1)])
            k = (k
                 - ((k >= 3) & (t0 > v)).astype(i32)
                 + ((k <= G3 - 1) & (t1 <= v)).astype(i32))
            kbuf_v[sl] = k
            plsc.addupdate_scatter(pos_v, [k], _ones_i32())
            nm1 = jnp.minimum(m1, v)
            nm2 = jnp.minimum(m2, jnp.maximum(m1, v))
            nx1 = jnp.maximum(mx1, v)
            nx2 = jnp.maximum(mx2, jnp.minimum(mx1, v))
            return a + v, nm1, nm2, nx1, nx2
        accv, m1v, m2v, x1v, x2v = lax.fori_loop(
            0, 64, tcnt,
            (jnp.zeros((16,), f32), jnp.full((16,), big, f32),
             jnp.full((16,), big, f32), jnp.full((16,), -big, f32),
             jnp.full((16,), -big, f32)), unroll=8)
        alpha = jnp.sum(accv) * np.float32(1.0 / 1024.0) - 0.5
        vmin = jnp.min(m1v)
        vmin2 = jnp.where(jnp.sum((m1v == vmin).astype(i32)) >= 2, vmin,
                          jnp.min(jnp.where(m1v == vmin, m2v, m1v)))
        vmax = jnp.max(x1v)
        vmax2 = jnp.where(jnp.sum((x1v == vmax).astype(i32)) >= 2, vmax,
                          jnp.max(jnp.where(x1v == vmax, x2v, x1v)))

        def cum_t(i, c):
            h = pos_v[pl.ds(i * 16, 16)]
            cs = plsc.cumsum(h)
            pos_v[pl.ds(i * 16, 16)] = cs - h + c
            return c + jnp.sum(h)
        lax.fori_loop(0, 49, cum_t, jnp.int32(0), unroll=4)

        def place_g(i, _):
            sl = pl.ds(i * 16, 16)
            k = kbuf_v[sl]
            b = plsc.load_gather(pos_v, [k])
            cnt, _unused = plsc.scan_count(k)
            plsc.store_scatter(tmp_v, [b + cnt.astype(i32) - 1], rowb_v[j, sl])
            plsc.addupdate_scatter(pos_v, [k], _ones_i32())
            return 0
        lax.fori_loop(0, 64, place_g, 0, unroll=4)

        def pmsm(i, carry):
            cx, cn = carry
            slf = pl.ds(i * 16, 16)
            gf = tmp_v[slf]
            srt_v[slf] = jnp.maximum(plsc.cummax(gf), cx)
            slr = pl.ds((63 - i) * 16, 16)
            gr = tmp_v[slr]
            rcm = -plsc.cummax(-lax.rev(gr, (0,)))
            sm_v[slr] = jnp.minimum(lax.rev(rcm, (0,)), cn)
            return jnp.maximum(cx, jnp.max(gf)), jnp.minimum(cn, jnp.min(gr))
        lax.fori_loop(0, 64, pmsm, (-big, big), unroll=4)

        def cdfe(i, _):
            sl = pl.ds(i * 16, 16)
            iq = pos_v[sl]
            lo = iq == 0
            hi = iq == N
            x0 = plsc.load_gather(srt_v, [jnp.clip(iq - 1, 0, N - 1)])
            x1 = plsc.load_gather(sm_v, [jnp.clip(iq, 0, N - 1)])
            x0 = jnp.where(lo, vmin, jnp.where(hi, vmax2, x0))
            x1 = jnp.where(lo, vmin2, jnp.where(hi, vmax, x1))
            y0 = jnp.clip(iq, 1, N - 1).astype(f32) * INV_N
            tq = tsort_v[sl]
            c = isort_v[sl] + y0 + INV_N * (tq - x0) / (x1 - x0)
            plsc.store_scatter(csrt_v, [perm_v[sl]], c)
            p = _floor_i32((c + alpha) * 256.0)
            p = jnp.clip(p + 1, 0, 256)
            r0 = plsc.load_gather(refg_v, [jnp.clip(p - 1, 0, 255)])
            r1 = plsc.load_gather(refg_v, [jnp.clip(p, 0, 255)])
            p = (p
                 - ((p >= 1) & (r0 - alpha > c)).astype(i32)
                 + ((p <= 255) & (r1 - alpha <= c)).astype(i32))
            plsc.addupdate_scatter(pos_v, [p + 2048], _ones_i32())
            return 0
        lax.fori_loop(0, 48, cdfe, 0, unroll=8)

        def cum_q(i, c):
            h = pos_v[pl.ds(i * 16 + 2048, 16)]
            cs = plsc.cumsum(h) + c
            pos_v[pl.ds(i * 16 + 2048, 16)] = cs
            return c + jnp.sum(h)
        lax.fori_loop(0, 16, cum_q, jnp.int32(0), unroll=2)

        def embe(i, _):
            sl = pl.ds(i * 16, 16)
            i2 = pos_v[pl.ds(i * 16 + 2048, 16)]
            ind = jnp.clip(i2 - 1, 0, G3 - 2)
            c0 = plsc.load_gather(csrt_v, [ind])
            c1 = plsc.load_gather(csrt_v, [ind + 1])
            rq = refg_v[sl]
            t = rq - alpha
            y0 = -1.0 + H_GRID * ind.astype(f32)
            embb_v[j, sl] = y0 + H_GRID * (t - c0) / (c1 - c0) - rq
            return 0
        lax.fori_loop(0, 16, embe, 0, unroll=4)
        return 0

    lax.fori_loop(0, 64, do_row, 0)
    pltpu.sync_copy(embb_v, out_hbm.at[pl.ds(wid * 64, 64)])


_sc_call = functools.partial(
    pl.kernel,
    out_type=jax.ShapeDtypeStruct((2048, REF_SIZE), f32),
    mesh=plsc.VectorSubcoreMesh(core_axis_name="c", subcore_axis_name="s"),
    compiler_params=pltpu.CompilerParams(needs_layout_passes=False),
    scratch_types=[
        pltpu.VMEM((G3,), f32),
        pltpu.VMEM((G3,), f32),
        pltpu.VMEM((REF_SIZE,), f32),
        pltpu.VMEM((G3,), i32),
        pltpu.VMEM((64, N), f32),
        pltpu.VMEM((N,), f32),
        pltpu.VMEM((N,), f32),
        pltpu.VMEM((N,), f32),
        pltpu.VMEM((N,), i32),
        pltpu.VMEM((G3,), f32),
        pltpu.VMEM((64, REF_SIZE), f32),
        pltpu.VMEM((2048 + 272,), i32),
    ],
)(_sc_embed)


def _loss_body(e1_ref, e2_ref, out_ref):
    d = jnp.abs(e2_ref[...] - e1_ref[...])
    m = jnp.minimum(d, 1.0 - d)
    s = jnp.sum(m * m, axis=2)
    loss = jnp.sqrt(s)
    out_ref[...] = jnp.broadcast_to(jnp.mean(loss, axis=1, keepdims=True), (8, 128))


def kernel(x1, x2, Z):
    Z0 = Z[:, :, 0]
    Z1 = Z[:, :, 1]

    q0, q1 = pl.pallas_call(
        _qr_body,
        out_shape=(jax.ShapeDtypeStruct((NUM_PROJ, D), f32),
                   jax.ShapeDtypeStruct((NUM_PROJ, D), f32)),
    )(Z0, Z1)

    X = jnp.stack([x1, x2]).reshape(16, 1024, D)
    ang = pl.pallas_call(
        _angles_body,
        out_shape=jax.ShapeDtypeStruct((16, NUM_PROJ, N), f32),
        grid=(16,),
        in_specs=[
            pl.BlockSpec((1, 1024, D), lambda i: (i, 0, 0)),
            pl.BlockSpec((NUM_PROJ, D), lambda i: (0, 0)),
            pl.BlockSpec((NUM_PROJ, D), lambda i: (0, 0)),
        ],
        out_specs=pl.BlockSpec((1, NUM_PROJ, N), lambda i: (i, 0, 0)),
    )(X, q0, q1)

    rows = ang.reshape(2048, N)

    xnew = jnp.linspace(-1.0, 2.0, G3).astype(f32)
    int_x = jnp.floor(xnew)
    rest_x = xnew - int_x
    perm = jnp.argsort(rest_x, stable=True)
    tsort = rest_x[perm]
    isort = int_x[perm]
    refg = jnp.linspace(0.0, 1.0, REF_SIZE + 1)[:-1].astype(f32)

    emb = _sc_call(rows, tsort, isort, refg, perm.astype(i32))

    e = emb.reshape(2, 8, NUM_PROJ, REF_SIZE)
    out = pl.pallas_call(
        _loss_body,
        out_shape=jax.ShapeDtypeStruct((8, 128), f32),
    )(e[0], e[1])
    return out[:, 0]
